# Initial kernel scaffold; baseline (speedup 1.0000x reference)
import jax
import jax.numpy as jnp
from jax.experimental import pallas as pl


def kernel(v, k, q, router_noise, g_v, b_v, g_k, b_k, g_q, b_q, Wq, Wk, Wv, Wo, bo, g_m, b_m, Wr, br, W1, b1, W2, b2, g_n, b_n):
    raise NotImplementedError("write your pallas kernel here")


# speedup vs baseline: 2.3797x; 2.3797x over previous
import functools

import jax
import jax.numpy as jnp
from jax.experimental import pallas as pl
from jax.experimental.pallas import tpu as pltpu

B, T, D = 1, 2048, 768
H = 8
DH = D // H
E = 8
TK = 2
HID = 512

_BF = jnp.bfloat16
_F32 = jnp.float32


def _mm(a, b):
    return jax.lax.dot_general(
        a.astype(_BF), b.astype(_BF),
        (((a.ndim - 1,), (0,)), ((), ())),
        preferred_element_type=_F32)


def _mm_t(a, b):
    return jax.lax.dot_general(
        a.astype(_BF), b.astype(_BF),
        (((0,), (0,)), ((), ())),
        preferred_element_type=_F32)


def _elu1(x):
    return jnp.where(x > 0, x + 1.0, jnp.exp(jnp.minimum(x, 0.0)))


def _ln(x, g, b):
    m = jnp.mean(x, axis=-1, keepdims=True)
    xc = x - m
    v = jnp.mean(xc * xc, axis=-1, keepdims=True)
    return xc * jax.lax.rsqrt(v + 1e-5) * g + b


_CHUNK = 512
_NC = T // _CHUNK


def _attn_kernel(v_ref, k_ref, q_ref, noise_ref,
                 g_v_ref, b_v_ref, g_k_ref, b_k_ref, g_q_ref, b_q_ref,
                 Wq_ref, Wk_ref, Wv_ref, Wo_ref, bo_ref,
                 g_m_ref, b_m_ref, Wr_ref, br_ref,
                 q2_out, xm_out, gates_out):
    kvs = [jnp.zeros((DH, DH), _F32) for _ in range(H)]
    k_sum = jnp.zeros((1, D), _F32)
    for c in range(_NC):
        rows = slice(c * _CHUNK, (c + 1) * _CHUNK)
        kn_c = _ln(k_ref[rows, :], g_k_ref[...], b_k_ref[...])
        vn_c = _ln(v_ref[rows, :], g_v_ref[...], b_v_ref[...])
        kh_c = _elu1(_mm(kn_c, Wk_ref[...]))
        vh_c = _mm(vn_c, Wv_ref[...])
        k_sum = k_sum + jnp.sum(kh_c, axis=0, keepdims=True)
        for h in range(H):
            sl = slice(h * DH, (h + 1) * DH)
            kvs[h] = kvs[h] + _mm_t(kh_c[:, sl], vh_c[:, sl])
    k_sum = k_sum + 1e-6

    for c in range(_NC):
        rows = slice(c * _CHUNK, (c + 1) * _CHUNK)
        q_c = q_ref[rows, :]
        qn_c = _ln(q_c, g_q_ref[...], b_q_ref[...])
        qh_c = _elu1(_mm(qn_c, Wq_ref[...]))
        outs = []
        for h in range(H):
            sl = slice(h * DH, (h + 1) * DH)
            oh = _mm(qh_c[:, sl], kvs[h])
            denom = _mm(qh_c[:, sl], k_sum[:, sl].T)
            outs.append(oh / denom)
        out_c = jnp.concatenate(outs, axis=1)
        attn_c = _mm(out_c, Wo_ref[...]) + bo_ref[...]
        q2_c = q_c + attn_c
        q2_out[rows, :] = q2_c
        xm_c = _ln(q2_c, g_m_ref[...], b_m_ref[...])
        xm_out[rows, :] = xm_c

        logits = _mm(xm_c, Wr_ref[...]) + br_ref[...] + noise_ref[rows, :] * 0.1
        lmax = jnp.max(logits, axis=-1, keepdims=True)
        ex = jnp.exp(logits - lmax)
        scores = ex / jnp.sum(ex, axis=-1, keepdims=True)
        iota = jax.lax.broadcasted_iota(jnp.int32, (_CHUNK, E), 1)
        m1 = jnp.max(scores, axis=-1, keepdims=True)
        i1 = jnp.min(jnp.where(scores == m1, iota, E), axis=-1, keepdims=True)
        mask1 = iota == i1
        s2 = jnp.where(mask1, -jnp.inf, scores)
        m2 = jnp.max(s2, axis=-1, keepdims=True)
        i2 = jnp.min(jnp.where(s2 == m2, iota, E), axis=-1, keepdims=True)
        mask2 = iota == i2
        gates_out[rows, :] = jnp.where(mask1 | mask2, scores, 0.0)


def _moe_kernel(xm_ref, gates_ref, q2_ref,
                W1_ref, b1_ref, W2_ref, b2_ref,
                g_n_ref, b_n_ref,
                out_ref, acc_ref):
    e = pl.program_id(0)
    xm = xm_ref[...]
    pre = _mm(xm, W1_ref[0]) + b1_ref[0]
    h = pre * 0.5 * (1.0 + jax.lax.erf(pre * 0.7071067811865476))
    oe = _mm(h, W2_ref[0]) + b2_ref[0]
    onehot = (jax.lax.broadcasted_iota(jnp.int32, (T, E), 1) == e)
    gate = jnp.sum(jnp.where(onehot, gates_ref[...], 0.0), axis=-1,
                   keepdims=True)
    contrib = gate * oe

    @pl.when(e == 0)
    def _():
        acc_ref[...] = contrib

    @pl.when(e > 0)
    def _():
        acc_ref[...] = acc_ref[...] + contrib

    @pl.when(e == E - 1)
    def _():
        moe = _ln(acc_ref[...] + xm, g_n_ref[...], b_n_ref[...])
        out_ref[...] = q2_ref[...] + moe


@functools.partial(jax.jit, static_argnames=())
def kernel(v, k, q, router_noise, g_v, b_v, g_k, b_k, g_q, b_q,
           Wq, Wk, Wv, Wo, bo, g_m, b_m, Wr, br, W1, b1, W2, b2, g_n, b_n):
    v2 = v.reshape(T, D)
    k2 = k.reshape(T, D)
    q2d = q.reshape(T, D)
    row = lambda x: x.reshape(1, -1)

    q2, xm, gates = pl.pallas_call(
        _attn_kernel,
        out_shape=(
            jax.ShapeDtypeStruct((T, D), _F32),
            jax.ShapeDtypeStruct((T, D), _F32),
            jax.ShapeDtypeStruct((T, E), _F32),
        ),
    )(v2, k2, q2d, router_noise,
      row(g_v), row(b_v), row(g_k), row(b_k), row(g_q), row(b_q),
      Wq, Wk, Wv, Wo, row(bo), row(g_m), row(b_m), Wr, row(br))

    out = pl.pallas_call(
        _moe_kernel,
        grid=(E,),
        in_specs=[
            pl.BlockSpec((T, D), lambda e: (0, 0)),
            pl.BlockSpec((T, E), lambda e: (0, 0)),
            pl.BlockSpec((T, D), lambda e: (0, 0)),
            pl.BlockSpec((1, D, HID), lambda e: (e, 0, 0)),
            pl.BlockSpec((1, 1, HID), lambda e: (e, 0, 0)),
            pl.BlockSpec((1, HID, D), lambda e: (e, 0, 0)),
            pl.BlockSpec((1, 1, D), lambda e: (e, 0, 0)),
            pl.BlockSpec((1, D), lambda e: (0, 0)),
            pl.BlockSpec((1, D), lambda e: (0, 0)),
        ],
        out_specs=pl.BlockSpec((T, D), lambda e: (0, 0)),
        out_shape=jax.ShapeDtypeStruct((T, D), _F32),
        scratch_shapes=[pltpu.VMEM((T, D), _F32)],
    )(xm, gates, q2, W1, b1.reshape(E, 1, HID), W2, b2.reshape(E, 1, D),
      row(g_n), row(b_n))

    return out.reshape(B, T, D)
